# baseline (speedup 1.0000x reference)
import jax
import jax.numpy as jnp
from jax import lax
from jax.experimental import pallas as pl
from jax.experimental.pallas import tpu as pltpu
from jax.experimental.pallas import tpu_sc as plsc

_SIZE = 1024
_ROWS = 32768
_N = 16384
_NC, _NS = 2, 16
_NW = _NC * _NS
_CHUNK = 16
_NCHUNK = _N // (_NW * _CHUNK)
_JCH = _SIZE // 16


def _gdot_body(tab_hbm, idx_hbm, w_hbm, b_hbm, out_hbm,
               idx_v, w_v, b_v, rows_v, vals_v, sems):
    wid = lax.axis_index("s") * _NC + lax.axis_index("c")
    base = wid * _NCHUNK
    pltpu.sync_copy(idx_hbm.at[pl.ds(base, _NCHUNK)], idx_v)
    pltpu.sync_copy(w_hbm, w_v)
    pltpu.sync_copy(b_hbm, b_v)
    bvec = b_v[...]

    def gather(c, buf):
        return pltpu.async_copy(tab_hbm.at[idx_v.at[c]], rows_v.at[buf],
                                sems.at[buf])

    lanes = lax.iota(jnp.int32, 16)
    rot_idx = [(lanes + sh) % 16 for sh in (8, 4, 2, 1)]
    gdn = lax.GatherDimensionNumbers(offset_dims=(), collapsed_slice_dims=(0,),
                                     start_index_map=(0,))

    def hsum(v):
        for idx in rot_idx:
            v = v + lax.gather(v, idx[:, None], gdn, slice_sizes=(1,),
                               mode=lax.GatherScatterMode.PROMISE_IN_BOUNDS)
        return v

    def dot_chunk(c, rows_ref):
        def jbody(j, accs):
            wj = w_v[pl.ds(j * 16, 16)]
            return tuple(
                accs[r] + rows_ref[r, pl.ds(j * 16, 16)] * wj
                for r in range(_CHUNK)
            )

        accs = lax.fori_loop(
            0, _JCH, jbody,
            tuple(jnp.zeros((16,), jnp.float32) for _ in range(_CHUNK)))
        v = jnp.zeros((16,), jnp.float32)
        for r in range(_CHUNK):
            v = jnp.where(lanes == r, hsum(accs[r]), v)
        vals_v[c, :] = v + bvec

    pending = gather(0, 0)
    for c in range(_NCHUNK):
        nxt = gather(c + 1, (c + 1) % 2) if c + 1 < _NCHUNK else None
        pending.wait()
        dot_chunk(c, rows_v.at[c % 2])
        pending = nxt
    pltpu.sync_copy(vals_v, out_hbm.at[pl.ds(base, _NCHUNK)])


def _sc_gather_dot(flat, idx2d, w_vec, b_vec):
    call = pl.kernel(
        _gdot_body,
        out_type=jax.ShapeDtypeStruct((_N // _CHUNK, _CHUNK), jnp.float32),
        mesh=plsc.VectorSubcoreMesh(core_axis_name="c", subcore_axis_name="s"),
        scratch_types=[
            pltpu.VMEM((_NCHUNK, _CHUNK), jnp.int32),
            pltpu.VMEM((_SIZE,), jnp.float32),
            pltpu.VMEM((16,), jnp.float32),
            pltpu.VMEM((2, _CHUNK, _SIZE), jnp.float32),
            pltpu.VMEM((_NCHUNK, _CHUNK), jnp.float32),
            pltpu.SemaphoreType.DMA((2,)),
        ],
    )
    return call(flat, idx2d, w_vec, b_vec)


def _loss_body(f_ref, t_ref, o_ref):
    f = f_ref[...]
    t = t_ref[...]
    val = jnp.sum(jnp.maximum(f, 0.0) - f * t + jnp.log1p(jnp.exp(-jnp.abs(f))))
    o_ref[...] = val.reshape(1, 1)


def _loss(final2d, targets2d):
    return pl.pallas_call(
        _loss_body,
        out_shape=jax.ShapeDtypeStruct((1, 1), jnp.float32),
    )(final2d, targets2d)


def kernel(rnn_output, indices, targets, W, b):
    flat = rnn_output.reshape(_ROWS, _SIZE)
    idx2d = indices.astype(jnp.int32).reshape(_N // _CHUNK, _CHUNK)
    w_vec = W.reshape(_SIZE)
    b_vec = jnp.broadcast_to(b, (16,))
    final2d = _sc_gather_dot(flat, idx2d, w_vec, b_vec)
    loss = _loss(final2d.reshape(128, 128), targets.reshape(128, 128))
    return final2d.reshape(_N), loss.reshape(())


# speedup vs baseline: 1.0052x; 1.0052x over previous
import jax
import jax.numpy as jnp
from jax import lax
from jax.experimental import pallas as pl
from jax.experimental.pallas import tpu as pltpu
from jax.experimental.pallas import tpu_sc as plsc

_SIZE = 1024
_ROWS = 32768
_N = 16384
_NC, _NS = 2, 16
_NW = _NC * _NS
_CHUNK = 16
_PERW = _N // _NW
_NCHUNK = _PERW // _CHUNK
_JCH = _SIZE // 16
_UNROLL = 4


def _gdot_body(tab_hbm, idx_hbm, w_hbm, b_hbm, out_hbm,
               idx_v, w_v, b_v, rows_v, vals_v, sems):
    wid = lax.axis_index("s") * _NC + lax.axis_index("c")
    base = wid * _NCHUNK
    pltpu.sync_copy(idx_hbm.at[pl.ds(base, _NCHUNK)], idx_v)
    pltpu.sync_copy(w_hbm, w_v)
    pltpu.sync_copy(b_hbm, b_v)
    bvec = b_v[...]

    def gather(c, buf):
        return pltpu.async_copy(tab_hbm.at[idx_v.at[c]], rows_v.at[buf],
                                sems.at[buf])

    lanes = lax.iota(jnp.int32, 16)
    rot_idx = [(lanes + sh) % 16 for sh in (8, 4, 2, 1)]
    gdn = lax.GatherDimensionNumbers(offset_dims=(), collapsed_slice_dims=(0,),
                                     start_index_map=(0,))

    def hsum(v):
        for idx in rot_idx:
            v = v + lax.gather(v, idx[:, None], gdn, slice_sizes=(1,),
                               mode=lax.GatherScatterMode.PROMISE_IN_BOUNDS)
        return v

    def dot_chunk(c, rows_ref):
        def jbody(j, accs):
            for u in range(_UNROLL):
                jj = j * _UNROLL + u
                wj = w_v[pl.ds(jj * 16, 16)]
                accs = tuple(
                    accs[r] + rows_ref[r, pl.ds(jj * 16, 16)] * wj
                    for r in range(_CHUNK)
                )
            return accs

        accs = lax.fori_loop(
            0, _JCH // _UNROLL, jbody,
            tuple(jnp.zeros((16,), jnp.float32) for _ in range(_CHUNK)))
        v = jnp.zeros((16,), jnp.float32)
        for r in range(_CHUNK):
            v = jnp.where(lanes == r, hsum(accs[r]), v)
        vals_v[pl.ds(c * _CHUNK, _CHUNK)] = v + bvec

    pending = gather(0, 0)
    for c in range(_NCHUNK):
        nxt = gather(c + 1, (c + 1) % 2) if c + 1 < _NCHUNK else None
        pending.wait()
        dot_chunk(c, rows_v.at[c % 2])
        pending = nxt
    pltpu.sync_copy(vals_v, out_hbm.at[pl.ds(wid * _PERW, _PERW)])


def _sc_gather_dot(flat, idx2d, w_vec, b_vec):
    call = pl.kernel(
        _gdot_body,
        out_type=jax.ShapeDtypeStruct((_N,), jnp.float32),
        mesh=plsc.VectorSubcoreMesh(core_axis_name="c", subcore_axis_name="s"),
        scratch_types=[
            pltpu.VMEM((_NCHUNK, _CHUNK), jnp.int32),
            pltpu.VMEM((_SIZE,), jnp.float32),
            pltpu.VMEM((16,), jnp.float32),
            pltpu.VMEM((2, _CHUNK, _SIZE), jnp.float32),
            pltpu.VMEM((_PERW,), jnp.float32),
            pltpu.SemaphoreType.DMA((2,)),
        ],
    )
    return call(flat, idx2d, w_vec, b_vec)


def _loss_body(f_ref, t_ref, o_ref):
    f = f_ref[...]
    t = t_ref[...]
    val = jnp.sum(jnp.maximum(f, 0.0) - f * t + jnp.log1p(jnp.exp(-jnp.abs(f))))
    o_ref[...] = val.reshape(1, 1)


def _loss(final, targets):
    return pl.pallas_call(
        _loss_body,
        out_shape=jax.ShapeDtypeStruct((1, 1), jnp.float32),
    )(final, targets)


def kernel(rnn_output, indices, targets, W, b):
    flat = rnn_output.reshape(_ROWS, _SIZE)
    idx2d = indices.astype(jnp.int32).reshape(_N // _CHUNK, _CHUNK)
    w_vec = W.reshape(_SIZE)
    b_vec = jnp.broadcast_to(b, (16,))
    final = _sc_gather_dot(flat, idx2d, w_vec, b_vec)
    loss = _loss(final, targets)
    return final, loss.reshape(())
